# Initial kernel scaffold; baseline (speedup 1.0000x reference)
import jax
import jax.numpy as jnp
from jax.experimental import pallas as pl


def kernel(x, keys_table, vals_table, fuse_w, fuse_b):
    raise NotImplementedError("write your pallas kernel here")


# speedup vs baseline: 5.0834x; 5.0834x over previous
import functools

import jax
import jax.numpy as jnp
from jax import lax
from jax.experimental import pallas as pl
from jax.experimental.pallas import tpu as pltpu
from jax.experimental.pallas import tpu_sc as plsc

D = 128
S = 200
V = 100000
NC, NS, L = 2, 16, 16
NW = NC * NS

_ROWS_BLK = 2000


def _transform_body(keys_ref, vals_ref, fw_ref, b_ref, tk_ref, tv_ref, c_ref):
    fw = fw_ref[...]
    w1 = fw[:, :D]
    dn = (((1,), (1,)), ((), ()))
    tk_ref[...] = lax.dot_general(keys_ref[...], w1, dn,
                                  preferred_element_type=jnp.float32)
    tv_ref[...] = lax.dot_general(vals_ref[...], w1, dn,
                                  preferred_element_type=jnp.float32)

    @pl.when(pl.program_id(0) == 0)
    def _():
        w2 = fw[:, D:]
        row = lax.broadcasted_iota(jnp.int32, (S, S), 0)
        col = lax.broadcasted_iota(jnp.int32, (S, S), 1)
        eye = jnp.where(row == col, 1.0, 0.0)
        c_ref[...] = lax.dot_general(eye, w2, dn,
                                     preferred_element_type=jnp.float32) + b_ref[...]


def _transform(keys_table, vals_table, fuse_w, fuse_b2d):
    grid = (V // _ROWS_BLK,)
    return pl.pallas_call(
        _transform_body,
        grid=grid,
        in_specs=[
            pl.BlockSpec((_ROWS_BLK, D), lambda i: (i, 0)),
            pl.BlockSpec((_ROWS_BLK, D), lambda i: (i, 0)),
            pl.BlockSpec((D, D + S), lambda i: (0, 0)),
            pl.BlockSpec((1, D), lambda i: (0, 0)),
        ],
        out_specs=[
            pl.BlockSpec((_ROWS_BLK, D), lambda i: (i, 0)),
            pl.BlockSpec((_ROWS_BLK, D), lambda i: (i, 0)),
            pl.BlockSpec((S, D), lambda i: (0, 0)),
        ],
        out_shape=[
            jax.ShapeDtypeStruct((V, D), jnp.float32),
            jax.ShapeDtypeStruct((V, D), jnp.float32),
            jax.ShapeDtypeStruct((S, D), jnp.float32),
        ],
    )(keys_table, vals_table, fuse_w, fuse_b2d)


def _gather_body(tk_hbm, tv_hbm, c_hbm, x_hbm, ko_hbm, vo_hbm,
                 idx_v, rows_k, rows_v, c_v, sem, rows_per_w):
    wid = lax.axis_index("s") * NC + lax.axis_index("c")
    pltpu.sync_copy(c_hbm, c_v)

    def row_body(r_local, carry):
        base = (wid * rows_per_w + r_local) * S
        pltpu.sync_copy(x_hbm.at[pl.ds(base, S)], idx_v)
        cps = [
            pltpu.async_copy(tk_hbm.at[idx_v.at[pl.ds(0, 104)]],
                             rows_k.at[pl.ds(0, 104)], sem),
            pltpu.async_copy(tk_hbm.at[idx_v.at[pl.ds(104, 96)]],
                             rows_k.at[pl.ds(104, 96)], sem),
            pltpu.async_copy(tv_hbm.at[idx_v.at[pl.ds(0, 104)]],
                             rows_v.at[pl.ds(0, 104)], sem),
            pltpu.async_copy(tv_hbm.at[idx_v.at[pl.ds(104, 96)]],
                             rows_v.at[pl.ds(104, 96)], sem),
        ]
        for cp in cps:
            cp.wait()

        def add_body(s_i, c2):
            for j in range(D // L):
                cvec = c_v[s_i, pl.ds(j * L, L)]
                plsc.addupdate(rows_k.at[s_i, pl.ds(j * L, L)], cvec)
                plsc.addupdate(rows_v.at[s_i, pl.ds(j * L, L)], cvec)
            return c2
        lax.fori_loop(0, S, add_body, 0)

        pltpu.sync_copy(rows_k, ko_hbm.at[pl.ds(base, S)])
        pltpu.sync_copy(rows_v, vo_hbm.at[pl.ds(base, S)])
        return carry

    lax.fori_loop(0, x_hbm.shape[0] // S // NW, row_body, 0)


def _gather(tk, tv, c, xf):
    n_tok = xf.shape[0]
    rows_per_w = n_tok // S // NW
    mesh = plsc.VectorSubcoreMesh(core_axis_name="c", subcore_axis_name="s")
    return pl.kernel(
        functools.partial(_gather_body, rows_per_w=rows_per_w),
        out_type=[
            jax.ShapeDtypeStruct((n_tok, D), jnp.float32),
            jax.ShapeDtypeStruct((n_tok, D), jnp.float32),
        ],
        mesh=mesh,
        scratch_types=[
            pltpu.VMEM((S,), jnp.int32),
            pltpu.VMEM((S, D), jnp.float32),
            pltpu.VMEM((S, D), jnp.float32),
            pltpu.VMEM((S, D), jnp.float32),
            pltpu.SemaphoreType.DMA,
        ],
    )(tk, tv, c, xf)


def kernel(x, keys_table, vals_table, fuse_w, fuse_b):
    tk, tv, c = _transform(keys_table, vals_table, fuse_w,
                           fuse_b.reshape(1, D))
    xf = x.reshape(-1).astype(jnp.int32)
    ko, vo = _gather(tk, tv, c, xf)
    b_sz, s_sz = x.shape
    return ko.reshape(b_sz, s_sz, D), vo.reshape(b_sz, s_sz, D)


# speedup vs baseline: 5.8071x; 1.1424x over previous
import functools

import jax
import jax.numpy as jnp
from jax import lax
from jax.experimental import pallas as pl
from jax.experimental.pallas import tpu as pltpu
from jax.experimental.pallas import tpu_sc as plsc

D = 128
S = 200
V = 100000
NC, NS, L = 2, 16, 16
NW = NC * NS

_ROWS_BLK = 2000


def _transform_body(keys_ref, vals_ref, fw_ref, b_ref, tk_ref, tv_ref, c_ref):
    fw = fw_ref[...]
    w1 = fw[:, :D]
    dn = (((1,), (1,)), ((), ()))
    tk_ref[...] = lax.dot_general(keys_ref[...], w1, dn,
                                  preferred_element_type=jnp.float32)
    tv_ref[...] = lax.dot_general(vals_ref[...], w1, dn,
                                  preferred_element_type=jnp.float32)

    @pl.when(pl.program_id(0) == 0)
    def _():
        w2 = fw[:, D:]
        row = lax.broadcasted_iota(jnp.int32, (S, S), 0)
        col = lax.broadcasted_iota(jnp.int32, (S, S), 1)
        eye = jnp.where(row == col, 1.0, 0.0)
        c_ref[...] = lax.dot_general(eye, w2, dn,
                                     preferred_element_type=jnp.float32) + b_ref[...]


def _transform(keys_table, vals_table, fuse_w, fuse_b2d):
    grid = (V // _ROWS_BLK,)
    return pl.pallas_call(
        _transform_body,
        grid=grid,
        in_specs=[
            pl.BlockSpec((_ROWS_BLK, D), lambda i: (i, 0)),
            pl.BlockSpec((_ROWS_BLK, D), lambda i: (i, 0)),
            pl.BlockSpec((D, D + S), lambda i: (0, 0)),
            pl.BlockSpec((1, D), lambda i: (0, 0)),
        ],
        out_specs=[
            pl.BlockSpec((_ROWS_BLK, D), lambda i: (i, 0)),
            pl.BlockSpec((_ROWS_BLK, D), lambda i: (i, 0)),
            pl.BlockSpec((S, D), lambda i: (0, 0)),
        ],
        out_shape=[
            jax.ShapeDtypeStruct((V, D), jnp.float32),
            jax.ShapeDtypeStruct((V, D), jnp.float32),
            jax.ShapeDtypeStruct((S, D), jnp.float32),
        ],
    )(keys_table, vals_table, fuse_w, fuse_b2d)


def _gather_body(tk_hbm, tv_hbm, c_hbm, x_hbm, ko_hbm, vo_hbm,
                 idx0, idx1, rk0, rk1, rv0, rv1, c_v,
                 sg0, sg1, sw0, sw1, rows_per_w):
    wid = lax.axis_index("s") * NC + lax.axis_index("c")
    base0 = wid * rows_per_w * S
    pltpu.sync_copy(c_hbm, c_v)

    idx = (idx0, idx1)
    rk = (rk0, rk1)
    rv = (rv0, rv1)
    sg = (sg0, sg1)
    sw = (sw0, sw1)

    def start_gather(b, r_local):
        base = base0 + r_local * S
        pltpu.sync_copy(x_hbm.at[pl.ds(base, S)], idx[b])
        pltpu.async_copy(tk_hbm.at[idx[b].at[pl.ds(0, 104)]],
                         rk[b].at[pl.ds(0, 104)], sg[b])
        pltpu.async_copy(tk_hbm.at[idx[b].at[pl.ds(104, 96)]],
                         rk[b].at[pl.ds(104, 96)], sg[b])
        pltpu.async_copy(tv_hbm.at[idx[b].at[pl.ds(0, 104)]],
                         rv[b].at[pl.ds(0, 104)], sg[b])
        pltpu.async_copy(tv_hbm.at[idx[b].at[pl.ds(104, 96)]],
                         rv[b].at[pl.ds(104, 96)], sg[b])

    def wait_gathers(b):
        pltpu.make_async_copy(tk_hbm.at[pl.ds(0, S)], rk[b], sg[b]).wait()
        pltpu.make_async_copy(tv_hbm.at[pl.ds(0, S)], rv[b], sg[b]).wait()

    def wait_writes(b):
        pltpu.make_async_copy(rk[b], ko_hbm.at[pl.ds(0, S)], sw[b]).wait()
        pltpu.make_async_copy(rv[b], vo_hbm.at[pl.ds(0, S)], sw[b]).wait()

    start_gather(0, 0)

    def row_pair_body(p, carry):
        for b in (0, 1):
            r_local = 2 * p + b
            base = base0 + r_local * S
            wait_gathers(b)

            def add_body(s_i, c2):
                for j in range(D // L):
                    cvec = c_v[s_i, pl.ds(j * L, L)]
                    plsc.addupdate(rk[b].at[s_i, pl.ds(j * L, L)], cvec)
                    plsc.addupdate(rv[b].at[s_i, pl.ds(j * L, L)], cvec)
                return c2
            lax.fori_loop(0, S, add_body, 0)

            pltpu.async_copy(rk[b], ko_hbm.at[pl.ds(base, S)], sw[b])
            pltpu.async_copy(rv[b], vo_hbm.at[pl.ds(base, S)], sw[b])

            @pl.when(r_local >= 1)
            def _():
                wait_writes(1 - b)

            @pl.when(r_local < rows_per_w - 1)
            def _():
                start_gather(1 - b, r_local + 1)
        return carry

    lax.fori_loop(0, rows_per_w // 2, row_pair_body, 0)
    wait_writes((rows_per_w - 1) % 2)


def _gather(tk, tv, c, xf):
    n_tok = xf.shape[0]
    rows_per_w = n_tok // S // NW
    mesh = plsc.VectorSubcoreMesh(core_axis_name="c", subcore_axis_name="s")
    return pl.kernel(
        functools.partial(_gather_body, rows_per_w=rows_per_w),
        out_type=[
            jax.ShapeDtypeStruct((n_tok, D), jnp.float32),
            jax.ShapeDtypeStruct((n_tok, D), jnp.float32),
        ],
        mesh=mesh,
        scratch_types=[
            pltpu.VMEM((S,), jnp.int32),
            pltpu.VMEM((S,), jnp.int32),
            pltpu.VMEM((S, D), jnp.float32),
            pltpu.VMEM((S, D), jnp.float32),
            pltpu.VMEM((S, D), jnp.float32),
            pltpu.VMEM((S, D), jnp.float32),
            pltpu.VMEM((S, D), jnp.float32),
            pltpu.SemaphoreType.DMA,
            pltpu.SemaphoreType.DMA,
            pltpu.SemaphoreType.DMA,
            pltpu.SemaphoreType.DMA,
        ],
    )(tk, tv, c, xf)


def kernel(x, keys_table, vals_table, fuse_w, fuse_b):
    tk, tv, c = _transform(keys_table, vals_table, fuse_w,
                           fuse_b.reshape(1, D))
    xf = x.reshape(-1).astype(jnp.int32)
    ko, vo = _gather(tk, tv, c, xf)
    b_sz, s_sz = x.shape
    return ko.reshape(b_sz, s_sz, D), vo.reshape(b_sz, s_sz, D)


# speedup vs baseline: 7.8827x; 1.3574x over previous
import functools

import jax
import jax.numpy as jnp
import numpy as np
from jax import lax
from jax.experimental import pallas as pl
from jax.experimental.pallas import tpu as pltpu
from jax.experimental.pallas import tpu_sc as plsc

D = 128
S = 200
V = 100000
DW = D // 2
NC, NS, L = 2, 16, 16
NW = NC * NS

_ROWS_BLK = 2000

_PERM_LO = np.concatenate([np.arange(32 * j, 32 * j + 16) for j in range(4)])
_PERM_HI = _PERM_LO + 16


def _rne_bf16_bits(f):
    u = lax.bitcast_convert_type(f, jnp.uint32)
    return (u + jnp.uint32(0x7FFF) + ((u >> 16) & jnp.uint32(1))) >> 16


def _pack_bf16_pair(lo, hi):
    w = _rne_bf16_bits(lo) | (_rne_bf16_bits(hi) << 16)
    return lax.bitcast_convert_type(w, jnp.int32)


def _transform_body(keys_ref, vals_ref, fwa_ref, fwb_ref, ba_ref, bb_ref,
                    tkv_ref, c_ref):
    fwa = fwa_ref[...]
    fwb = fwb_ref[...]
    dn = (((1,), (1,)), ((), ()))
    mm = functools.partial(lax.dot_general, dimension_numbers=dn,
                           preferred_element_type=jnp.float32)
    keys = keys_ref[...]
    vals = vals_ref[...]
    pk = _pack_bf16_pair(mm(keys, fwa[:, :D]), mm(keys, fwb[:, :D]))
    pv = _pack_bf16_pair(mm(vals, fwa[:, :D]), mm(vals, fwb[:, :D]))
    tkv_ref[...] = jnp.concatenate([pk, pv], axis=1)

    @pl.when(pl.program_id(0) == 0)
    def _():
        row = lax.broadcasted_iota(jnp.int32, (S, S), 0)
        col = lax.broadcasted_iota(jnp.int32, (S, S), 1)
        eye = jnp.where(row == col, 1.0, 0.0)
        pc = _pack_bf16_pair(mm(eye, fwa[:, D:]) + ba_ref[...],
                             mm(eye, fwb[:, D:]) + bb_ref[...])
        c_ref[...] = jnp.concatenate([pc, pc], axis=1)


def _transform(keys_table, vals_table, fwa, fwb, ba2d, bb2d):
    grid = (V // _ROWS_BLK,)
    return pl.pallas_call(
        _transform_body,
        grid=grid,
        in_specs=[
            pl.BlockSpec((_ROWS_BLK, D), lambda i: (i, 0)),
            pl.BlockSpec((_ROWS_BLK, D), lambda i: (i, 0)),
            pl.BlockSpec((DW, D + S), lambda i: (0, 0)),
            pl.BlockSpec((DW, D + S), lambda i: (0, 0)),
            pl.BlockSpec((1, DW), lambda i: (0, 0)),
            pl.BlockSpec((1, DW), lambda i: (0, 0)),
        ],
        out_specs=[
            pl.BlockSpec((_ROWS_BLK, D), lambda i: (i, 0)),
            pl.BlockSpec((S, D), lambda i: (0, 0)),
        ],
        out_shape=[
            jax.ShapeDtypeStruct((V, D), jnp.int32),
            jax.ShapeDtypeStruct((S, D), jnp.int32),
        ],
    )(keys_table, vals_table, fwa, fwb, ba2d, bb2d)


def _gather_body(tkv_hbm, c_hbm, x_hbm, ko_hbm, vo_hbm,
                 idx0, idx1, rkv0, rkv1, c_v, stk, stv,
                 sg0, sg1, sws, rows_per_w):
    wid = lax.axis_index("s") * NC + lax.axis_index("c")
    base0 = wid * rows_per_w * S
    pltpu.sync_copy(c_hbm, c_v)

    idx = (idx0, idx1)
    rkv = (rkv0, rkv1)
    sg = (sg0, sg1)

    def start_gather(b, r_local):
        base = base0 + r_local * S
        pltpu.sync_copy(x_hbm.at[pl.ds(base, S)], idx[b])
        pltpu.async_copy(tkv_hbm.at[idx[b].at[pl.ds(0, 104)]],
                         rkv[b].at[pl.ds(0, 104)], sg[b])
        pltpu.async_copy(tkv_hbm.at[idx[b].at[pl.ds(104, 96)]],
                         rkv[b].at[pl.ds(104, 96)], sg[b])

    def wait_gathers(b):
        pltpu.make_async_copy(tkv_hbm.at[pl.ds(0, S)], rkv[b], sg[b]).wait()

    def wait_st_writes():
        pltpu.make_async_copy(stk, ko_hbm.at[pl.ds(0, S)], sws).wait()
        pltpu.make_async_copy(stv, vo_hbm.at[pl.ds(0, S)], sws).wait()

    start_gather(0, 0)

    def row_pair_body(p, carry):
        for b in (0, 1):
            r_local = 2 * p + b
            base = base0 + r_local * S

            @pl.when(r_local < rows_per_w - 1)
            def _():
                start_gather(1 - b, r_local + 1)

            wait_gathers(b)

            @pl.when(r_local >= 1)
            def _():
                wait_st_writes()

            def unpack2(w):
                lo = lax.bitcast_convert_type(w << 16, jnp.float32)
                hi = lax.bitcast_convert_type(
                    w & jnp.int32(-65536), jnp.float32)
                return lo, hi

            def add_body(s_i, c2):
                for j in range(D // 32):
                    ksl = pl.ds(16 * j, 16)
                    vsl = pl.ds(DW + 16 * j, 16)
                    ca, cb = unpack2(c_v[s_i, ksl])
                    ka, kb = unpack2(rkv[b][s_i, ksl])
                    va, vb = unpack2(rkv[b][s_i, vsl])
                    stk[s_i, pl.ds(32 * j, 16)] = ka + ca
                    stk[s_i, pl.ds(32 * j + 16, 16)] = kb + cb
                    stv[s_i, pl.ds(32 * j, 16)] = va + ca
                    stv[s_i, pl.ds(32 * j + 16, 16)] = vb + cb
                return c2
            lax.fori_loop(0, S, add_body, 0)

            pltpu.async_copy(stk, ko_hbm.at[pl.ds(base, S)], sws)
            pltpu.async_copy(stv, vo_hbm.at[pl.ds(base, S)], sws)
        return carry

    lax.fori_loop(0, rows_per_w // 2, row_pair_body, 0)
    wait_st_writes()


def _gather(tkv, c, xf):
    n_tok = xf.shape[0]
    rows_per_w = n_tok // S // NW
    mesh = plsc.VectorSubcoreMesh(core_axis_name="c", subcore_axis_name="s")
    return pl.kernel(
        functools.partial(_gather_body, rows_per_w=rows_per_w),
        out_type=[
            jax.ShapeDtypeStruct((n_tok, D), jnp.float32),
            jax.ShapeDtypeStruct((n_tok, D), jnp.float32),
        ],
        mesh=mesh,
        scratch_types=[
            pltpu.VMEM((S,), jnp.int32),
            pltpu.VMEM((S,), jnp.int32),
            pltpu.VMEM((S, D), jnp.int32),
            pltpu.VMEM((S, D), jnp.int32),
            pltpu.VMEM((S, D), jnp.int32),
            pltpu.VMEM((S, D), jnp.float32),
            pltpu.VMEM((S, D), jnp.float32),
            pltpu.SemaphoreType.DMA,
            pltpu.SemaphoreType.DMA,
            pltpu.SemaphoreType.DMA,
        ],
    )(tkv, c, xf)


def kernel(x, keys_table, vals_table, fuse_w, fuse_b):
    lo = jnp.asarray(_PERM_LO)
    hi = jnp.asarray(_PERM_HI)
    tkv, c = _transform(keys_table, vals_table,
                        fuse_w[lo, :], fuse_w[hi, :],
                        fuse_b[lo].reshape(1, DW), fuse_b[hi].reshape(1, DW))
    xf = x.reshape(-1).astype(jnp.int32)
    ko, vo = _gather(tkv, c, xf)
    b_sz, s_sz = x.shape
    return ko.reshape(b_sz, s_sz, D), vo.reshape(b_sz, s_sz, D)


# speedup vs baseline: 8.1027x; 1.0279x over previous
import functools

import jax
import jax.numpy as jnp
import numpy as np
from jax import lax
from jax.experimental import pallas as pl
from jax.experimental.pallas import tpu as pltpu
from jax.experimental.pallas import tpu_sc as plsc

D = 128
S = 200
V = 100000
DW = D // 2
NC, NS, L = 2, 16, 16
NW = NC * NS

_ROWS_BLK = 2000

_PERM_LO = np.concatenate([np.arange(32 * j, 32 * j + 16) for j in range(4)])
_PERM_HI = _PERM_LO + 16


def _bf16_bits(f):
    b = lax.bitcast_convert_type(f.astype(jnp.bfloat16), jnp.uint16)
    return b.astype(jnp.uint32)


def _pack_bf16_pair(lo, hi):
    w = _bf16_bits(lo) | (_bf16_bits(hi) << 16)
    return lax.bitcast_convert_type(w, jnp.int32)


def _transform_body(keys_ref, vals_ref, fwa_ref, fwb_ref, ba_ref, bb_ref,
                    tkv_ref, c_ref):
    fwa = fwa_ref[...]
    fwb = fwb_ref[...]
    dn = (((1,), (1,)), ((), ()))
    mm = functools.partial(lax.dot_general, dimension_numbers=dn,
                           preferred_element_type=jnp.float32)
    keys = keys_ref[...]
    vals = vals_ref[...]
    pk = _pack_bf16_pair(mm(keys, fwa[:, :D]), mm(keys, fwb[:, :D]))
    pv = _pack_bf16_pair(mm(vals, fwa[:, :D]), mm(vals, fwb[:, :D]))
    tkv_ref[...] = jnp.concatenate([pk, pv], axis=1)

    @pl.when(pl.program_id(0) == 0)
    def _():
        row = lax.broadcasted_iota(jnp.int32, (S, S), 0)
        col = lax.broadcasted_iota(jnp.int32, (S, S), 1)
        eye = jnp.where(row == col, 1.0, 0.0)
        pc = _pack_bf16_pair(mm(eye, fwa[:, D:]) + ba_ref[...],
                             mm(eye, fwb[:, D:]) + bb_ref[...])
        c_ref[...] = jnp.concatenate([pc, pc], axis=1)


def _transform(keys_table, vals_table, fwa, fwb, ba2d, bb2d):
    grid = (V // _ROWS_BLK,)
    return pl.pallas_call(
        _transform_body,
        grid=grid,
        in_specs=[
            pl.BlockSpec((_ROWS_BLK, D), lambda i: (i, 0)),
            pl.BlockSpec((_ROWS_BLK, D), lambda i: (i, 0)),
            pl.BlockSpec((DW, D + S), lambda i: (0, 0)),
            pl.BlockSpec((DW, D + S), lambda i: (0, 0)),
            pl.BlockSpec((1, DW), lambda i: (0, 0)),
            pl.BlockSpec((1, DW), lambda i: (0, 0)),
        ],
        out_specs=[
            pl.BlockSpec((_ROWS_BLK, D), lambda i: (i, 0)),
            pl.BlockSpec((S, D), lambda i: (0, 0)),
        ],
        out_shape=[
            jax.ShapeDtypeStruct((V, D), jnp.int32),
            jax.ShapeDtypeStruct((S, D), jnp.int32),
        ],
    )(keys_table, vals_table, fwa, fwb, ba2d, bb2d)


def _gather_body(tkv_hbm, c_hbm, x_hbm, ko_hbm, vo_hbm,
                 idx0, idx1, rkv0, rkv1, c_v, stk, stv,
                 sg0, sg1, sws, rows_per_w):
    wid = lax.axis_index("s") * NC + lax.axis_index("c")
    base0 = wid * rows_per_w * S
    pltpu.sync_copy(c_hbm, c_v)

    idx = (idx0, idx1)
    rkv = (rkv0, rkv1)
    sg = (sg0, sg1)

    def start_gather(b, r_local):
        base = base0 + r_local * S
        pltpu.sync_copy(x_hbm.at[pl.ds(base, S)], idx[b])
        pltpu.async_copy(tkv_hbm.at[idx[b].at[pl.ds(0, 104)]],
                         rkv[b].at[pl.ds(0, 104)], sg[b])
        pltpu.async_copy(tkv_hbm.at[idx[b].at[pl.ds(104, 96)]],
                         rkv[b].at[pl.ds(104, 96)], sg[b])

    def wait_gathers(b):
        pltpu.make_async_copy(tkv_hbm.at[pl.ds(0, S)], rkv[b], sg[b]).wait()

    def wait_st_writes():
        pltpu.make_async_copy(stk, ko_hbm.at[pl.ds(0, S)], sws).wait()
        pltpu.make_async_copy(stv, vo_hbm.at[pl.ds(0, S)], sws).wait()

    start_gather(0, 0)

    def row_pair_body(p, carry):
        for b in (0, 1):
            r_local = 2 * p + b
            base = base0 + r_local * S

            @pl.when(r_local < rows_per_w - 1)
            def _():
                start_gather(1 - b, r_local + 1)

            wait_gathers(b)

            @pl.when(r_local >= 1)
            def _():
                wait_st_writes()

            def unpack2(w):
                lo = lax.bitcast_convert_type(w << 16, jnp.float32)
                hi = lax.bitcast_convert_type(
                    w & jnp.int32(-65536), jnp.float32)
                return lo, hi

            def add_body(s_i, c2):
                for j in range(D // 32):
                    ksl = pl.ds(16 * j, 16)
                    vsl = pl.ds(DW + 16 * j, 16)
                    ca, cb = unpack2(c_v[s_i, ksl])
                    ka, kb = unpack2(rkv[b][s_i, ksl])
                    va, vb = unpack2(rkv[b][s_i, vsl])
                    stk[s_i, pl.ds(32 * j, 16)] = ka + ca
                    stk[s_i, pl.ds(32 * j + 16, 16)] = kb + cb
                    stv[s_i, pl.ds(32 * j, 16)] = va + ca
                    stv[s_i, pl.ds(32 * j + 16, 16)] = vb + cb
                return c2
            lax.fori_loop(0, S, add_body, 0)

            pltpu.async_copy(stk, ko_hbm.at[pl.ds(base, S)], sws)
            pltpu.async_copy(stv, vo_hbm.at[pl.ds(base, S)], sws)
        return carry

    lax.fori_loop(0, rows_per_w // 2, row_pair_body, 0)
    wait_st_writes()


def _gather(tkv, c, xf):
    n_tok = xf.shape[0]
    rows_per_w = n_tok // S // NW
    mesh = plsc.VectorSubcoreMesh(core_axis_name="c", subcore_axis_name="s")
    return pl.kernel(
        functools.partial(_gather_body, rows_per_w=rows_per_w),
        out_type=[
            jax.ShapeDtypeStruct((n_tok, D), jnp.float32),
            jax.ShapeDtypeStruct((n_tok, D), jnp.float32),
        ],
        mesh=mesh,
        scratch_types=[
            pltpu.VMEM((S,), jnp.int32),
            pltpu.VMEM((S,), jnp.int32),
            pltpu.VMEM((S, D), jnp.int32),
            pltpu.VMEM((S, D), jnp.int32),
            pltpu.VMEM((S, D), jnp.int32),
            pltpu.VMEM((S, D), jnp.float32),
            pltpu.VMEM((S, D), jnp.float32),
            pltpu.SemaphoreType.DMA,
            pltpu.SemaphoreType.DMA,
            pltpu.SemaphoreType.DMA,
        ],
    )(tkv, c, xf)


def kernel(x, keys_table, vals_table, fuse_w, fuse_b):
    lo = jnp.asarray(_PERM_LO)
    hi = jnp.asarray(_PERM_HI)
    tkv, c = _transform(keys_table, vals_table,
                        fuse_w[lo, :], fuse_w[hi, :],
                        fuse_b[lo].reshape(1, DW), fuse_b[hi].reshape(1, DW))
    xf = x.reshape(-1).astype(jnp.int32)
    ko, vo = _gather(tkv, c, xf)
    b_sz, s_sz = x.shape
    return ko.reshape(b_sz, s_sz, D), vo.reshape(b_sz, s_sz, D)
